# baseline (speedup 1.0000x reference)
import functools

import jax
import jax.numpy as jnp
from jax import lax
from jax.experimental import pallas as pl
from jax.experimental.pallas import tpu as pltpu
from jax.experimental.pallas import tpu_sc as plsc

E = 64
IDX_W = 128
FIRE = 4


def _sc_gather(table, idx2d):
    R = idx2d.shape[0]
    EP = table.shape[1]
    info = plsc.get_sparse_core_info()
    nw = info.num_cores * info.num_subcores
    rows_per_w = R // nw
    n_blocks = rows_per_w // FIRE
    mesh = plsc.VectorSubcoreMesh(core_axis_name="c", subcore_axis_name="s")

    @functools.partial(
        pl.kernel,
        out_type=jax.ShapeDtypeStruct((R * IDX_W, EP), jnp.float32),
        mesh=mesh,
        scratch_types=[
            pltpu.VMEM((FIRE, IDX_W), jnp.int32),
            pltpu.VMEM((FIRE * IDX_W, EP), jnp.float32),
            pltpu.SemaphoreType.DMA,
        ],
        compiler_params=pltpu.CompilerParams(use_tc_tiling_on_sc=False),
    )
    def k(table_hbm, idx_hbm, out_hbm, idx_v, rows_v, sem):
        wid = lax.axis_index("s") * info.num_cores + lax.axis_index("c")
        row0 = wid * rows_per_w

        def body(j, carry):
            base = row0 + j * FIRE
            pltpu.sync_copy(idx_hbm.at[pl.ds(base, FIRE)], idx_v)
            copies = [
                pltpu.async_copy(
                    table_hbm.at[idx_v.at[b]],
                    rows_v.at[pl.ds(b * IDX_W, IDX_W)],
                    sem,
                )
                for b in range(FIRE)
            ]
            for c in copies:
                c.wait()
            pltpu.sync_copy(rows_v, out_hbm.at[pl.ds(base * IDX_W, FIRE * IDX_W)])
            return carry

        lax.fori_loop(0, n_blocks, body, 0)

    return k(table, idx2d)


def _tc_combine(gathered, pos_q, stacked, eos_row, B, S, *, tb=32):
    T = tb * S

    def body(g_ref, q_ref, tab_ref, eos_ref, out_ref):
        iota = lax.broadcasted_iota(jnp.int32, (T, 192), 1)
        q = q_ref[...]
        p0 = q & 63
        p1 = (q >> 6) & 63
        p2 = q >> 12
        mh = ((p0 == iota).astype(jnp.float32)
              + ((p1 + 64) == iota).astype(jnp.float32)
              + ((p2 + 128) == iota).astype(jnp.float32))
        pe = jnp.dot(mh, tab_ref[...], preferred_element_type=jnp.float32)
        pe = pe.reshape(tb, S, E)
        eos_blk = jnp.broadcast_to(eos_ref[...].reshape(1, 1, E), (tb, 1, E))
        pe_next = jnp.concatenate([pe[:, 1:, :], eos_blk], axis=1)
        out = g_ref[:, :E].reshape(tb, S, E) + pe + pe_next
        out_ref[...] = out.reshape(T, E)

    return pl.pallas_call(
        body,
        grid=(B // tb,),
        in_specs=[
            pl.BlockSpec((T, E), lambda i: (i, 0)),
            pl.BlockSpec((T, 1), lambda i: (i, 0)),
            pl.BlockSpec((192, E), lambda i: (0, 0)),
            pl.BlockSpec((1, E), lambda i: (0, 0)),
        ],
        out_specs=pl.BlockSpec((T, E), lambda i: (i, 0)),
        out_shape=jax.ShapeDtypeStruct((B * S, E), jnp.float32),
    )(gathered, pos_q, stacked, eos_row)


def kernel(value, depth, position, value_table, spatial_tables, la_tables, eos):
    del depth, la_tables
    B, S = value.shape
    N = B * S
    idx2d = value.reshape(N // IDX_W, IDX_W).astype(jnp.int32)
    gathered = _sc_gather(value_table, idx2d)
    weights = jnp.array([1, 64, 4096], jnp.int32)
    pos_q = (position.astype(jnp.int32) * weights).sum(axis=2).reshape(N, 1)
    stacked = spatial_tables.reshape(3 * spatial_tables.shape[1], E)
    eos_row = eos.reshape(1, E)
    out = _tc_combine(gathered, pos_q, stacked, eos_row, B, S)
    return out.reshape(B, S, E)


# speedup vs baseline: 1.1452x; 1.1452x over previous
import functools

import jax
import jax.numpy as jnp
from jax import lax
from jax.experimental import pallas as pl
from jax.experimental.pallas import tpu as pltpu
from jax.experimental.pallas import tpu_sc as plsc

E = 64
IDX_W = 128
FIRE = 4


def _sc_gather(table, idx2d):
    R = idx2d.shape[0]
    EP = table.shape[1]
    info = plsc.get_sparse_core_info()
    nw = info.num_cores * info.num_subcores
    rows_per_w = R // nw
    n_blocks = rows_per_w // FIRE
    mesh = plsc.VectorSubcoreMesh(core_axis_name="c", subcore_axis_name="s")

    @functools.partial(
        pl.kernel,
        out_type=jax.ShapeDtypeStruct((R * IDX_W, EP), jnp.float32),
        mesh=mesh,
        scratch_types=[
            pltpu.VMEM((FIRE, IDX_W), jnp.int32),
            pltpu.VMEM((FIRE * IDX_W, EP), jnp.float32),
            pltpu.SemaphoreType.DMA,
        ],
        compiler_params=pltpu.CompilerParams(use_tc_tiling_on_sc=True),
    )
    def k(table_hbm, idx_hbm, out_hbm, idx_v, rows_v, sem):
        wid = lax.axis_index("s") * info.num_cores + lax.axis_index("c")
        row0 = wid * rows_per_w

        def body(j, carry):
            base = row0 + j * FIRE
            pltpu.sync_copy(idx_hbm.at[pl.ds(base, FIRE)], idx_v)
            copies = [
                pltpu.async_copy(
                    table_hbm.at[idx_v.at[b]],
                    rows_v.at[pl.ds(b * IDX_W, IDX_W)],
                    sem,
                )
                for b in range(FIRE)
            ]
            for c in copies:
                c.wait()
            pltpu.sync_copy(rows_v, out_hbm.at[pl.ds(base * IDX_W, FIRE * IDX_W)])
            return carry

        lax.fori_loop(0, n_blocks, body, 0)

    return k(table, idx2d)


def _tc_combine(gathered, pos_q, stacked, eos_row, B, S, *, tb=32):
    T = tb * S

    def body(g_ref, q_ref, tab_ref, eos_ref, out_ref):
        iota = lax.broadcasted_iota(jnp.int32, (T, 192), 1)
        q = q_ref[...]
        p0 = q & 63
        p1 = (q >> 6) & 63
        p2 = q >> 12
        mh = ((p0 == iota).astype(jnp.float32)
              + ((p1 + 64) == iota).astype(jnp.float32)
              + ((p2 + 128) == iota).astype(jnp.float32))
        pe = jnp.dot(mh, tab_ref[...], preferred_element_type=jnp.float32)
        pe = pe.reshape(tb, S, E)
        eos_blk = jnp.broadcast_to(eos_ref[...].reshape(1, 1, E), (tb, 1, E))
        pe_next = jnp.concatenate([pe[:, 1:, :], eos_blk], axis=1)
        out = g_ref[:, :E].reshape(tb, S, E) + pe + pe_next
        out_ref[...] = out.reshape(T, E)

    return pl.pallas_call(
        body,
        grid=(B // tb,),
        in_specs=[
            pl.BlockSpec((T, 128), lambda i: (i, 0)),
            pl.BlockSpec((T, 1), lambda i: (i, 0)),
            pl.BlockSpec((192, E), lambda i: (0, 0)),
            pl.BlockSpec((1, E), lambda i: (0, 0)),
        ],
        out_specs=pl.BlockSpec((T, E), lambda i: (i, 0)),
        out_shape=jax.ShapeDtypeStruct((B * S, E), jnp.float32),
    )(gathered, pos_q, stacked, eos_row)


def kernel(value, depth, position, value_table, spatial_tables, la_tables, eos):
    del depth, la_tables
    B, S = value.shape
    N = B * S
    idx2d = value.reshape(N // IDX_W, IDX_W).astype(jnp.int32)
    table_p = jnp.pad(value_table, ((0, 0), (0, 128 - E)))
    gathered = _sc_gather(table_p, idx2d)
    weights = jnp.array([1, 64, 4096], jnp.int32)
    pos_q = (position.astype(jnp.int32) * weights).sum(axis=2).reshape(N, 1)
    stacked = spatial_tables.reshape(3 * spatial_tables.shape[1], E)
    eos_row = eos.reshape(1, E)
    out = _tc_combine(gathered, pos_q, stacked, eos_row, B, S)
    return out.reshape(B, S, E)
